# Initial kernel scaffold; baseline (speedup 1.0000x reference)
import jax
import jax.numpy as jnp
from jax.experimental import pallas as pl


def kernel(x, inW, inb, ln_g, ln_b, xpW, xpb, dpW, dpb, A, C, gW, gb, oW, ob, hln_g, hln_b, h1W, h1b, h2W, h2b):
    raise NotImplementedError("write your pallas kernel here")


# speedup vs baseline: 9.3518x; 9.3518x over previous
import jax
import jax.numpy as jnp
from jax.experimental import pallas as pl
from jax.experimental.pallas import tpu as pltpu

_B, _T, _F, _H, _S = 32, 2048, 64, 512, 64
_TC = 64
_NC = _T // _TC
_R = _TC * _B
_SUB = 256
_EPS = 1e-5


def _layer_body(h_ref, lng_ref, lnb_ref, xpW_ref, xpb_ref, dpW_ref, dpb_ref,
                At_ref, Ct_ref, gW_ref, gb_ref, oW_ref, ob_ref,
                o_ref, state_ref, xp_s, dl_s, gate_s, st_s):
    j = pl.program_id(0)

    @pl.when(j == 0)
    def _():
        state_ref[...] = jnp.zeros_like(state_ref)

    for r0 in range(0, _R, _SUB):
        rows = h_ref[r0:r0 + _SUB, :]
        mu = jnp.mean(rows, axis=-1, keepdims=True)
        ctr = rows - mu
        var = jnp.mean(ctr * ctr, axis=-1, keepdims=True)
        xn = ctr * jax.lax.rsqrt(var + _EPS) * lng_ref[...] + lnb_ref[...]
        xp_s[r0:r0 + _SUB, :] = (
            jnp.dot(xn, xpW_ref[...], preferred_element_type=jnp.float32)
            + xpb_ref[...])
        dl_s[r0:r0 + _SUB, :] = jax.nn.sigmoid(
            jnp.dot(xn, dpW_ref[...], preferred_element_type=jnp.float32)
            + dpb_ref[...])
        gate_s[r0:r0 + _SUB, :] = jax.nn.sigmoid(
            jnp.dot(xn, gW_ref[...], preferred_element_type=jnp.float32)
            + gb_ref[...])

    st = state_ref[...]
    A = At_ref[...]
    for t in range(_TC):
        xt = xp_s[t * _B:(t + 1) * _B, :]
        dt = dl_s[t * _B:(t + 1) * _B, :]
        sA = jnp.dot(st, A, preferred_element_type=jnp.float32)
        st = (1.0 - dt) * st + dt * (sA + xt)
        st_s[t * _B:(t + 1) * _B, :] = st
    state_ref[...] = st

    for r0 in range(0, _R, _SUB):
        sC = jnp.dot(st_s[r0:r0 + _SUB, :], Ct_ref[...],
                     preferred_element_type=jnp.float32)
        out = jnp.dot(gate_s[r0:r0 + _SUB, :] * sC, oW_ref[...],
                      preferred_element_type=jnp.float32)
        o_ref[r0:r0 + _SUB, :] = out + ob_ref[...] + h_ref[r0:r0 + _SUB, :]


def _layer(h2, lng, lnb, xpW, xpb, dpW, dpb, At, Ct, gW, gb, oW, ob):
    full = lambda s: pl.BlockSpec(s, lambda j: (0,) * len(s))
    return pl.pallas_call(
        _layer_body,
        grid=(_NC,),
        in_specs=[
            pl.BlockSpec((_R, _H), lambda j: (j, 0)),
            full((1, _H)), full((1, _H)),
            full((_H, _S)), full((1, _S)),
            full((_H, _S)), full((1, _S)),
            full((_S, _S)), full((_S, _H)),
            full((_H, _H)), full((1, _H)),
            full((_H, _H)), full((1, _H)),
        ],
        out_specs=pl.BlockSpec((_R, _H), lambda j: (j, 0)),
        out_shape=jax.ShapeDtypeStruct((_T * _B, _H), jnp.float32),
        scratch_shapes=[
            pltpu.VMEM((_B, _S), jnp.float32),
            pltpu.VMEM((_R, _S), jnp.float32),
            pltpu.VMEM((_R, _S), jnp.float32),
            pltpu.VMEM((_R, _H), jnp.float32),
            pltpu.VMEM((_R, _S), jnp.float32),
        ],
        compiler_params=pltpu.CompilerParams(
            dimension_semantics=("arbitrary",),
            vmem_limit_bytes=56 * 1024 * 1024,
        ),
        name="mamba_layer",
    )(h2, lng, lnb, xpW, xpb, dpW, dpb, At, Ct, gW, gb, oW, ob)


def _inproj_body(x_ref, w_ref, b_ref, o_ref):
    o_ref[...] = (jnp.dot(x_ref[...], w_ref[...],
                          preferred_element_type=jnp.float32) + b_ref[...])


def _inproj(xt, inW, inb):
    rows = 4096
    return pl.pallas_call(
        _inproj_body,
        grid=(_T * _B // rows,),
        in_specs=[
            pl.BlockSpec((rows, _F), lambda j: (j, 0)),
            pl.BlockSpec((_F, _H), lambda j: (0, 0)),
            pl.BlockSpec((1, _H), lambda j: (0, 0)),
        ],
        out_specs=pl.BlockSpec((rows, _H), lambda j: (j, 0)),
        out_shape=jax.ShapeDtypeStruct((_T * _B, _H), jnp.float32),
        compiler_params=pltpu.CompilerParams(
            dimension_semantics=("parallel",),
            vmem_limit_bytes=56 * 1024 * 1024,
        ),
        name="mamba_inproj",
    )(xt, inW, inb)


def _erf(z):
    s = jnp.where(z < 0, -1.0, 1.0)
    a = jnp.abs(z)
    t = 1.0 / (1.0 + 0.3275911 * a)
    p = t * (0.254829592 + t * (-0.284496736 + t * (1.421413741
        + t * (-1.453152027 + t * 1.061405429))))
    return s * (1.0 - p * jnp.exp(-a * a))


def _head_body(h_ref, g_ref, b_ref, w1_ref, b1_ref, w2_ref, b2_ref, o_ref):
    rows = h_ref[...]
    mu = jnp.mean(rows, axis=-1, keepdims=True)
    ctr = rows - mu
    var = jnp.mean(ctr * ctr, axis=-1, keepdims=True)
    y = ctr * jax.lax.rsqrt(var + _EPS) * g_ref[...] + b_ref[...]
    y = jnp.dot(y, w1_ref[...], preferred_element_type=jnp.float32) + b1_ref[...]
    y = y * 0.5 * (1.0 + _erf(y * 0.7071067811865476))
    o_ref[...] = (jnp.dot(y, w2_ref[...], preferred_element_type=jnp.float32)
                  + b2_ref[...])


def _head(last, hln_g, hln_b, h1W, h1b, h2W, h2b):
    return pl.pallas_call(
        _head_body,
        out_shape=jax.ShapeDtypeStruct((_B, 1), jnp.float32),
        name="mamba_head",
    )(last, hln_g, hln_b, h1W, h1b, h2W, h2b)


def kernel(x, inW, inb, ln_g, ln_b, xpW, xpb, dpW, dpb, A, C, gW, gb,
           oW, ob, hln_g, hln_b, h1W, h1b, h2W, h2b):
    L = ln_g.shape[0]
    xt = jnp.transpose(x, (1, 0, 2)).reshape(_T * _B, _F)
    h = _inproj(xt, inW, inb.reshape(1, _H))
    for i in range(L):
        h = _layer(h,
                   ln_g[i].reshape(1, _H), ln_b[i].reshape(1, _H),
                   xpW[i], xpb[i].reshape(1, _S),
                   dpW[i], dpb[i].reshape(1, _S),
                   A[i].T, C[i].T,
                   gW[i], gb[i].reshape(1, _H),
                   oW[i], ob[i].reshape(1, _H))
    last = h[-_B:, :]
    return _head(last, hln_g.reshape(1, _H), hln_b.reshape(1, _H),
                 h1W, h1b.reshape(1, _H // 2), h2W, h2b.reshape(1, 1))


# speedup vs baseline: 10.7031x; 1.1445x over previous
import jax
import jax.numpy as jnp
from jax.experimental import pallas as pl
from jax.experimental.pallas import tpu as pltpu

_B, _T, _F, _H, _S = 32, 2048, 64, 512, 64
_TC = 64
_NC = _T // _TC
_R = _TC * _B
_SUB = 256
_EPS = 1e-5


def _layer_body(h_ref, lng_ref, lnb_ref, xpW_ref, xpb_ref, dpW_ref, dpb_ref,
                At_ref, Ct_ref, gW_ref, gb_ref, oW_ref, ob_ref,
                o_ref, state_ref, gate_a, gate_b):
    j = pl.program_id(0)

    @pl.when(j == 0)
    def _():
        state_ref[...] = jnp.zeros_like(state_ref)

    nsub = _R // _SUB
    spt = _SUB // _B
    gbufs = (gate_a, gate_b)

    xn_v = [None] * nsub
    xp_v = [None] * nsub
    dl_v = [None] * nsub
    sts_v = [None] * nsub
    st_sl = [None] * _TC

    def _ln_op(s):
        r0 = s * _SUB
        rows = h_ref[r0:r0 + _SUB, :]
        mu = jnp.mean(rows, axis=-1, keepdims=True)
        ctr = rows - mu
        var = jnp.mean(ctr * ctr, axis=-1, keepdims=True)
        xn = ctr * jax.lax.rsqrt(var + _EPS) * lng_ref[...] + lnb_ref[...]
        xn_v[s] = xn.astype(jnp.bfloat16)

    def _xp_op(s):
        xp_v[s] = (jnp.dot(xn_v[s], xpW_ref[...],
                           preferred_element_type=jnp.float32) + xpb_ref[...])

    def _dl_op(s):
        dl_v[s] = jax.nn.sigmoid(
            jnp.dot(xn_v[s], dpW_ref[...], preferred_element_type=jnp.float32)
            + dpb_ref[...])

    def _gate_op(s):
        g = jax.nn.sigmoid(
            jnp.dot(xn_v[s], gW_ref[...], preferred_element_type=jnp.float32)
            + gb_ref[...])
        gbufs[s % 2][...] = g.astype(jnp.bfloat16)

    def _post_op(s):
        r0 = s * _SUB
        sC = jnp.dot(sts_v[s], Ct_ref[...], preferred_element_type=jnp.float32)
        prod = gbufs[s % 2][...] * sC.astype(jnp.bfloat16)
        out = jnp.dot(prod, oW_ref[...], preferred_element_type=jnp.float32)
        o_ref[r0:r0 + _SUB, :] = out + ob_ref[...] + h_ref[r0:r0 + _SUB, :]

    st = state_ref[...]
    A = At_ref[...]
    _ln_op(0)
    _xp_op(0)
    _dl_op(0)
    _gate_op(0)
    for t in range(_TC):
        s, k = divmod(t, spt)
        if k == 0 and s >= 1:
            _post_op(s - 1)
        if s + 1 < nsub:
            if k == 1:
                _ln_op(s + 1)
            elif k == 2:
                _xp_op(s + 1)
            elif k == 3:
                _dl_op(s + 1)
            elif k == 4:
                _gate_op(s + 1)
        xt = xp_v[s][k * _B:(k + 1) * _B, :]
        dt = dl_v[s][k * _B:(k + 1) * _B, :]
        sA = jnp.dot(st, A, preferred_element_type=jnp.float32)
        st = (1.0 - dt) * st + dt * (sA + xt)
        st_sl[t] = st
        if k == spt - 1:
            sts_v[s] = jnp.concatenate(
                st_sl[s * spt:(s + 1) * spt], axis=0).astype(jnp.bfloat16)
    _post_op(nsub - 1)
    state_ref[...] = st


def _layer(h2, lng, lnb, xpW, xpb, dpW, dpb, At, Ct, gW, gb, oW, ob):
    full = lambda s: pl.BlockSpec(s, lambda j: (0,) * len(s))
    return pl.pallas_call(
        _layer_body,
        grid=(_NC,),
        in_specs=[
            pl.BlockSpec((_R, _H), lambda j: (j, 0)),
            full((1, _H)), full((1, _H)),
            full((_H, _S)), full((1, _S)),
            full((_H, _S)), full((1, _S)),
            full((_S, _S)), full((_S, _H)),
            full((_H, _H)), full((1, _H)),
            full((_H, _H)), full((1, _H)),
        ],
        out_specs=pl.BlockSpec((_R, _H), lambda j: (j, 0)),
        out_shape=jax.ShapeDtypeStruct((_T * _B, _H), jnp.float32),
        scratch_shapes=[
            pltpu.VMEM((_B, _S), jnp.float32),
            pltpu.VMEM((_SUB, _H), jnp.bfloat16),
            pltpu.VMEM((_SUB, _H), jnp.bfloat16),
        ],
        compiler_params=pltpu.CompilerParams(
            dimension_semantics=("arbitrary",),
            vmem_limit_bytes=56 * 1024 * 1024,
        ),
        name="mamba_layer",
    )(h2, lng, lnb, xpW, xpb, dpW, dpb, At, Ct, gW, gb, oW, ob)


def _inproj_body(x_ref, w_ref, b_ref, o_ref):
    o_ref[...] = (jnp.dot(x_ref[...], w_ref[...],
                          preferred_element_type=jnp.float32) + b_ref[...])


def _inproj(xt, inW, inb):
    rows = 4096
    return pl.pallas_call(
        _inproj_body,
        grid=(_T * _B // rows,),
        in_specs=[
            pl.BlockSpec((rows, _F), lambda j: (j, 0)),
            pl.BlockSpec((_F, _H), lambda j: (0, 0)),
            pl.BlockSpec((1, _H), lambda j: (0, 0)),
        ],
        out_specs=pl.BlockSpec((rows, _H), lambda j: (j, 0)),
        out_shape=jax.ShapeDtypeStruct((_T * _B, _H), jnp.float32),
        compiler_params=pltpu.CompilerParams(
            dimension_semantics=("parallel",),
            vmem_limit_bytes=56 * 1024 * 1024,
        ),
        name="mamba_inproj",
    )(xt, inW, inb)


def _erf(z):
    s = jnp.where(z < 0, -1.0, 1.0)
    a = jnp.abs(z)
    t = 1.0 / (1.0 + 0.3275911 * a)
    p = t * (0.254829592 + t * (-0.284496736 + t * (1.421413741
        + t * (-1.453152027 + t * 1.061405429))))
    return s * (1.0 - p * jnp.exp(-a * a))


def _head_body(h_ref, g_ref, b_ref, w1_ref, b1_ref, w2_ref, b2_ref, o_ref):
    rows = h_ref[...]
    mu = jnp.mean(rows, axis=-1, keepdims=True)
    ctr = rows - mu
    var = jnp.mean(ctr * ctr, axis=-1, keepdims=True)
    y = ctr * jax.lax.rsqrt(var + _EPS) * g_ref[...] + b_ref[...]
    y = jnp.dot(y, w1_ref[...], preferred_element_type=jnp.float32) + b1_ref[...]
    y = y * 0.5 * (1.0 + _erf(y * 0.7071067811865476))
    o_ref[...] = (jnp.dot(y, w2_ref[...], preferred_element_type=jnp.float32)
                  + b2_ref[...])


def _head(last, hln_g, hln_b, h1W, h1b, h2W, h2b):
    return pl.pallas_call(
        _head_body,
        out_shape=jax.ShapeDtypeStruct((_B, 1), jnp.float32),
        name="mamba_head",
    )(last, hln_g, hln_b, h1W, h1b, h2W, h2b)


def kernel(x, inW, inb, ln_g, ln_b, xpW, xpb, dpW, dpb, A, C, gW, gb,
           oW, ob, hln_g, hln_b, h1W, h1b, h2W, h2b):
    L = ln_g.shape[0]
    xt = jnp.transpose(x, (1, 0, 2)).reshape(_T * _B, _F)
    h = _inproj(xt, inW, inb.reshape(1, _H))
    bf = jnp.bfloat16
    for i in range(L):
        h = _layer(h,
                   ln_g[i].reshape(1, _H), ln_b[i].reshape(1, _H),
                   xpW[i].astype(bf), xpb[i].reshape(1, _S),
                   dpW[i].astype(bf), dpb[i].reshape(1, _S),
                   A[i].T, C[i].T.astype(bf),
                   gW[i].astype(bf), gb[i].reshape(1, _H),
                   oW[i].astype(bf), ob[i].reshape(1, _H))
    last = h[-_B:, :]
    return _head(last, hln_g.reshape(1, _H), hln_b.reshape(1, _H),
                 h1W, h1b.reshape(1, _H // 2), h2W, h2b.reshape(1, 1))


# speedup vs baseline: 11.0294x; 1.0305x over previous
import jax
import jax.numpy as jnp
from jax.experimental import pallas as pl
from jax.experimental.pallas import tpu as pltpu

_B, _T, _F, _H, _S = 32, 2048, 64, 512, 64
_TC = 128
_NC = _T // _TC
_R = _TC * _B
_SUB = 256
_EPS = 1e-5


def _layer_body(h_ref, lng_ref, lnb_ref, xpW_ref, xpb_ref, dpW_ref, dpb_ref,
                At_ref, Ct_ref, gW_ref, gb_ref, oW_ref, ob_ref,
                o_ref, state_ref, gate_a, gate_b):
    j = pl.program_id(0)

    @pl.when(j == 0)
    def _():
        state_ref[...] = jnp.zeros_like(state_ref)

    nsub = _R // _SUB
    spt = _SUB // _B
    gbufs = (gate_a, gate_b)

    xn_v = [None] * nsub
    xp_v = [None] * nsub
    dl_v = [None] * nsub
    sts_v = [None] * nsub
    st_sl = [None] * _TC

    def _ln_op(s):
        r0 = s * _SUB
        rows = h_ref[r0:r0 + _SUB, :]
        mu = jnp.mean(rows, axis=-1, keepdims=True)
        ctr = rows - mu
        var = jnp.mean(ctr * ctr, axis=-1, keepdims=True)
        xn = ctr * jax.lax.rsqrt(var + _EPS) * lng_ref[...] + lnb_ref[...]
        xn_v[s] = xn.astype(jnp.bfloat16)

    def _xp_op(s):
        xp_v[s] = (jnp.dot(xn_v[s], xpW_ref[...],
                           preferred_element_type=jnp.float32) + xpb_ref[...])

    def _dl_op(s):
        dl_v[s] = jax.nn.sigmoid(
            jnp.dot(xn_v[s], dpW_ref[...], preferred_element_type=jnp.float32)
            + dpb_ref[...])

    def _gate_op(s):
        g = jax.nn.sigmoid(
            jnp.dot(xn_v[s], gW_ref[...], preferred_element_type=jnp.float32)
            + gb_ref[...])
        gbufs[s % 2][...] = g.astype(jnp.bfloat16)

    def _post_op(s):
        r0 = s * _SUB
        sC = jnp.dot(sts_v[s], Ct_ref[...], preferred_element_type=jnp.float32)
        prod = gbufs[s % 2][...] * sC.astype(jnp.bfloat16)
        out = jnp.dot(prod, oW_ref[...], preferred_element_type=jnp.float32)
        o_ref[r0:r0 + _SUB, :] = out + ob_ref[...] + h_ref[r0:r0 + _SUB, :]

    st = state_ref[...]
    A = At_ref[...]
    _ln_op(0)
    _xp_op(0)
    _dl_op(0)
    _gate_op(0)
    for t in range(_TC):
        s, k = divmod(t, spt)
        if k == 0 and s >= 1:
            _post_op(s - 1)
        if s + 1 < nsub:
            if k == 1:
                _ln_op(s + 1)
            elif k == 2:
                _xp_op(s + 1)
            elif k == 3:
                _dl_op(s + 1)
            elif k == 4:
                _gate_op(s + 1)
        xt = xp_v[s][k * _B:(k + 1) * _B, :]
        dt = dl_v[s][k * _B:(k + 1) * _B, :]
        sA = jnp.dot(st, A, preferred_element_type=jnp.float32)
        st = (1.0 - dt) * st + dt * (sA + xt)
        st_sl[t] = st
        if k == spt - 1:
            sts_v[s] = jnp.concatenate(
                st_sl[s * spt:(s + 1) * spt], axis=0).astype(jnp.bfloat16)
    _post_op(nsub - 1)
    state_ref[...] = st


def _layer(h2, lng, lnb, xpW, xpb, dpW, dpb, At, Ct, gW, gb, oW, ob):
    full = lambda s: pl.BlockSpec(s, lambda j: (0,) * len(s))
    return pl.pallas_call(
        _layer_body,
        grid=(_NC,),
        in_specs=[
            pl.BlockSpec((_R, _H), lambda j: (j, 0)),
            full((1, _H)), full((1, _H)),
            full((_H, _S)), full((1, _S)),
            full((_H, _S)), full((1, _S)),
            full((_S, _S)), full((_S, _H)),
            full((_H, _H)), full((1, _H)),
            full((_H, _H)), full((1, _H)),
        ],
        out_specs=pl.BlockSpec((_R, _H), lambda j: (j, 0)),
        out_shape=jax.ShapeDtypeStruct((_T * _B, _H), jnp.float32),
        scratch_shapes=[
            pltpu.VMEM((_B, _S), jnp.float32),
            pltpu.VMEM((_SUB, _H), jnp.bfloat16),
            pltpu.VMEM((_SUB, _H), jnp.bfloat16),
        ],
        compiler_params=pltpu.CompilerParams(
            dimension_semantics=("arbitrary",),
            vmem_limit_bytes=56 * 1024 * 1024,
        ),
        name="mamba_layer",
    )(h2, lng, lnb, xpW, xpb, dpW, dpb, At, Ct, gW, gb, oW, ob)


def _inproj_body(x_ref, w_ref, b_ref, o_ref):
    o_ref[...] = (jnp.dot(x_ref[...], w_ref[...],
                          preferred_element_type=jnp.float32) + b_ref[...])


def _inproj(xt, inW, inb):
    rows = 4096
    return pl.pallas_call(
        _inproj_body,
        grid=(_T * _B // rows,),
        in_specs=[
            pl.BlockSpec((rows, _F), lambda j: (j, 0)),
            pl.BlockSpec((_F, _H), lambda j: (0, 0)),
            pl.BlockSpec((1, _H), lambda j: (0, 0)),
        ],
        out_specs=pl.BlockSpec((rows, _H), lambda j: (j, 0)),
        out_shape=jax.ShapeDtypeStruct((_T * _B, _H), jnp.float32),
        compiler_params=pltpu.CompilerParams(
            dimension_semantics=("parallel",),
            vmem_limit_bytes=56 * 1024 * 1024,
        ),
        name="mamba_inproj",
    )(xt, inW, inb)


def _erf(z):
    s = jnp.where(z < 0, -1.0, 1.0)
    a = jnp.abs(z)
    t = 1.0 / (1.0 + 0.3275911 * a)
    p = t * (0.254829592 + t * (-0.284496736 + t * (1.421413741
        + t * (-1.453152027 + t * 1.061405429))))
    return s * (1.0 - p * jnp.exp(-a * a))


def _head_body(h_ref, g_ref, b_ref, w1_ref, b1_ref, w2_ref, b2_ref, o_ref):
    rows = h_ref[...]
    mu = jnp.mean(rows, axis=-1, keepdims=True)
    ctr = rows - mu
    var = jnp.mean(ctr * ctr, axis=-1, keepdims=True)
    y = ctr * jax.lax.rsqrt(var + _EPS) * g_ref[...] + b_ref[...]
    y = jnp.dot(y, w1_ref[...], preferred_element_type=jnp.float32) + b1_ref[...]
    y = y * 0.5 * (1.0 + _erf(y * 0.7071067811865476))
    o_ref[...] = (jnp.dot(y, w2_ref[...], preferred_element_type=jnp.float32)
                  + b2_ref[...])


def _head(last, hln_g, hln_b, h1W, h1b, h2W, h2b):
    return pl.pallas_call(
        _head_body,
        out_shape=jax.ShapeDtypeStruct((_B, 1), jnp.float32),
        name="mamba_head",
    )(last, hln_g, hln_b, h1W, h1b, h2W, h2b)


def kernel(x, inW, inb, ln_g, ln_b, xpW, xpb, dpW, dpb, A, C, gW, gb,
           oW, ob, hln_g, hln_b, h1W, h1b, h2W, h2b):
    L = ln_g.shape[0]
    xt = jnp.transpose(x, (1, 0, 2)).reshape(_T * _B, _F)
    h = _inproj(xt, inW, inb.reshape(1, _H))
    bf = jnp.bfloat16
    for i in range(L):
        h = _layer(h,
                   ln_g[i].reshape(1, _H), ln_b[i].reshape(1, _H),
                   xpW[i].astype(bf), xpb[i].reshape(1, _S),
                   dpW[i].astype(bf), dpb[i].reshape(1, _S),
                   A[i].T, C[i].T.astype(bf),
                   gW[i].astype(bf), gb[i].reshape(1, _H),
                   oW[i].astype(bf), ob[i].reshape(1, _H))
    last = h[-_B:, :]
    return _head(last, hln_g.reshape(1, _H), hln_b.reshape(1, _H),
                 h1W, h1b.reshape(1, _H // 2), h2W, h2b.reshape(1, 1))


# speedup vs baseline: 14.9968x; 1.3597x over previous
import jax
import jax.numpy as jnp
from jax.experimental import pallas as pl
from jax.experimental.pallas import tpu as pltpu

_B, _T, _F, _H, _S = 32, 2048, 64, 512, 64
_TC = 64
_NC = _T // _TC
_R = _TC * _B
_SUB = 256
_NSUB = _R // _SUB
_SPT = _SUB // _B
_EPS = 1e-5


class _LayerCtx:
    def __init__(self, wrefs, gbufs, read_rows, write_rows, st0):
        (self.lng, self.lnb, self.xpW, self.xpb, self.dpW, self.dpb,
         self.At, self.Ct, self.gW, self.gb, self.oW, self.ob) = wrefs
        self.gbufs = gbufs
        self.read_rows = read_rows
        self.write_rows = write_rows
        self.st = st0
        self.xn_v = [None] * _NSUB
        self.xp_v = [None] * _NSUB
        self.dl_v = [None] * _NSUB
        self.sts_v = [None] * _NSUB
        self.st_sl = [None] * _TC

    def ln_op(self, s):
        rows = self.read_rows(s * _SUB, _SUB)
        mu = jnp.mean(rows, axis=-1, keepdims=True)
        ctr = rows - mu
        var = jnp.mean(ctr * ctr, axis=-1, keepdims=True)
        xn = (ctr * jax.lax.rsqrt(var + _EPS) * self.lng[...]
              + self.lnb[...])
        self.xn_v[s] = xn.astype(jnp.bfloat16)

    def xp_op(self, s):
        self.xp_v[s] = (jnp.dot(self.xn_v[s], self.xpW[...],
                                preferred_element_type=jnp.float32)
                        + self.xpb[...])

    def dl_op(self, s):
        self.dl_v[s] = jax.nn.sigmoid(
            jnp.dot(self.xn_v[s], self.dpW[...],
                    preferred_element_type=jnp.float32) + self.dpb[...])

    def gate_op(self, s):
        g = jax.nn.sigmoid(
            jnp.dot(self.xn_v[s], self.gW[...],
                    preferred_element_type=jnp.float32) + self.gb[...])
        self.gbufs[s % 2][...] = g.astype(jnp.bfloat16)

    def post_op(self, s):
        sC = jnp.dot(self.sts_v[s], self.Ct[...],
                     preferred_element_type=jnp.float32)
        prod = self.gbufs[s % 2][...] * sC.astype(jnp.bfloat16)
        out = jnp.dot(prod, self.oW[...], preferred_element_type=jnp.float32)
        res = self.read_rows(s * _SUB, _SUB)
        self.write_rows(s * _SUB, out + self.ob[...] + res)

    def step(self, t):
        s, k = divmod(t, _SPT)
        if k == 0 and s >= 1:
            self.post_op(s - 1)
        if s + 1 < _NSUB:
            if k == 1:
                self.ln_op(s + 1)
            elif k == 2:
                self.xp_op(s + 1)
            elif k == 3:
                self.dl_op(s + 1)
            elif k == 4:
                self.gate_op(s + 1)
        xt = self.xp_v[s][k * _B:(k + 1) * _B, :]
        dt = self.dl_v[s][k * _B:(k + 1) * _B, :]
        sA = jnp.dot(self.st, self.At[...], preferred_element_type=jnp.float32)
        self.st = (1.0 - dt) * self.st + dt * (sA + xt)
        self.st_sl[t] = self.st
        if k == _SPT - 1:
            self.sts_v[s] = jnp.concatenate(
                self.st_sl[s * _SPT:(s + 1) * _SPT],
                axis=0).astype(jnp.bfloat16)


def _pair_body(h_ref, *rest):
    wa = rest[0:12]
    wb = rest[12:24]
    (o_ref, ring_ref, state_a, state_b,
     ga0, ga1, gb0, gb1) = rest[24:]
    j = pl.program_id(0)
    pw = jax.lax.rem(j, 2)
    pr = jax.lax.rem(j + 1, 2)

    ctx_a = _LayerCtx(
        wa, (ga0, ga1),
        lambda r0, n: h_ref[r0:r0 + n, :],
        lambda r0, v: ring_ref.__setitem__(
            (pw, pl.ds(r0, v.shape[0]), slice(None)), v),
        jnp.where(j > 0, state_a[...], 0.0))
    ctx_b = _LayerCtx(
        wb, (gb0, gb1),
        lambda r0, n: ring_ref[pr, r0:r0 + n, :],
        lambda r0, v: o_ref.__setitem__(
            (pl.ds(r0, v.shape[0]), slice(None)), v),
        jnp.where(j > 1, state_b[...], 0.0))

    for ctx in (ctx_b, ctx_a):
        ctx.ln_op(0)
        ctx.xp_op(0)
        ctx.dl_op(0)
        ctx.gate_op(0)
    for t in range(_TC):
        ctx_b.step(t)
        ctx_a.step(t)
    ctx_b.post_op(_NSUB - 1)
    ctx_a.post_op(_NSUB - 1)
    state_a[...] = ctx_a.st
    state_b[...] = ctx_b.st


def _pair(h2, wlist_a, wlist_b):
    full = lambda s: pl.BlockSpec(s, lambda j: (0,) * len(s))
    wspecs = [
        full((1, _H)), full((1, _H)),
        full((_H, _S)), full((1, _S)),
        full((_H, _S)), full((1, _S)),
        full((_S, _S)), full((_S, _H)),
        full((_H, _H)), full((1, _H)),
        full((_H, _H)), full((1, _H)),
    ]
    return pl.pallas_call(
        _pair_body,
        grid=(_NC + 1,),
        in_specs=[pl.BlockSpec((_R, _H),
                               lambda j: (jnp.minimum(j, _NC - 1), 0))]
                 + wspecs + wspecs,
        out_specs=pl.BlockSpec((_R, _H), lambda j: (jnp.maximum(j - 1, 0), 0)),
        out_shape=jax.ShapeDtypeStruct((_T * _B, _H), jnp.float32),
        scratch_shapes=[
            pltpu.VMEM((2, _R, _H), jnp.float32),
            pltpu.VMEM((_B, _S), jnp.float32),
            pltpu.VMEM((_B, _S), jnp.float32),
            pltpu.VMEM((_SUB, _H), jnp.bfloat16),
            pltpu.VMEM((_SUB, _H), jnp.bfloat16),
            pltpu.VMEM((_SUB, _H), jnp.bfloat16),
            pltpu.VMEM((_SUB, _H), jnp.bfloat16),
        ],
        compiler_params=pltpu.CompilerParams(
            dimension_semantics=("arbitrary",),
            vmem_limit_bytes=56 * 1024 * 1024,
        ),
        name="mamba_pair",
    )(h2, *wlist_a, *wlist_b)


def _inproj_body(x_ref, w_ref, b_ref, o_ref):
    o_ref[...] = (jnp.dot(x_ref[...], w_ref[...],
                          preferred_element_type=jnp.float32) + b_ref[...])


def _inproj(xt, inW, inb):
    rows = 4096
    return pl.pallas_call(
        _inproj_body,
        grid=(_T * _B // rows,),
        in_specs=[
            pl.BlockSpec((rows, _F), lambda j: (j, 0)),
            pl.BlockSpec((_F, _H), lambda j: (0, 0)),
            pl.BlockSpec((1, _H), lambda j: (0, 0)),
        ],
        out_specs=pl.BlockSpec((rows, _H), lambda j: (j, 0)),
        out_shape=jax.ShapeDtypeStruct((_T * _B, _H), jnp.float32),
        compiler_params=pltpu.CompilerParams(
            dimension_semantics=("parallel",),
            vmem_limit_bytes=56 * 1024 * 1024,
        ),
        name="mamba_inproj",
    )(xt, inW, inb)


def _erf(z):
    s = jnp.where(z < 0, -1.0, 1.0)
    a = jnp.abs(z)
    t = 1.0 / (1.0 + 0.3275911 * a)
    p = t * (0.254829592 + t * (-0.284496736 + t * (1.421413741
        + t * (-1.453152027 + t * 1.061405429))))
    return s * (1.0 - p * jnp.exp(-a * a))


def _head_body(h_ref, g_ref, b_ref, w1_ref, b1_ref, w2_ref, b2_ref, o_ref):
    rows = h_ref[...]
    mu = jnp.mean(rows, axis=-1, keepdims=True)
    ctr = rows - mu
    var = jnp.mean(ctr * ctr, axis=-1, keepdims=True)
    y = ctr * jax.lax.rsqrt(var + _EPS) * g_ref[...] + b_ref[...]
    y = jnp.dot(y, w1_ref[...], preferred_element_type=jnp.float32) + b1_ref[...]
    y = y * 0.5 * (1.0 + _erf(y * 0.7071067811865476))
    o_ref[...] = (jnp.dot(y, w2_ref[...], preferred_element_type=jnp.float32)
                  + b2_ref[...])


def _head(last, hln_g, hln_b, h1W, h1b, h2W, h2b):
    return pl.pallas_call(
        _head_body,
        out_shape=jax.ShapeDtypeStruct((_B, 1), jnp.float32),
        name="mamba_head",
    )(last, hln_g, hln_b, h1W, h1b, h2W, h2b)


def kernel(x, inW, inb, ln_g, ln_b, xpW, xpb, dpW, dpb, A, C, gW, gb,
           oW, ob, hln_g, hln_b, h1W, h1b, h2W, h2b):
    L = ln_g.shape[0]
    bf = jnp.bfloat16

    def wlist(i):
        return [ln_g[i].reshape(1, _H), ln_b[i].reshape(1, _H),
                xpW[i].astype(bf), xpb[i].reshape(1, _S),
                dpW[i].astype(bf), dpb[i].reshape(1, _S),
                A[i].T, C[i].T.astype(bf),
                gW[i].astype(bf), gb[i].reshape(1, _H),
                oW[i].astype(bf), ob[i].reshape(1, _H)]

    xt = jnp.transpose(x, (1, 0, 2)).reshape(_T * _B, _F)
    h = _inproj(xt, inW, inb.reshape(1, _H))
    for i in range(0, L, 2):
        h = _pair(h, wlist(i), wlist(i + 1))
    last = h[-_B:, :]
    return _head(last, hln_g.reshape(1, _H), hln_b.reshape(1, _H),
                 h1W, h1b.reshape(1, _H // 2), h2W, h2b.reshape(1, 1))
